# baseline (speedup 1.0000x reference)
import functools
import jax
import jax.numpy as jnp
from jax import lax
from jax.experimental import pallas as pl
from jax.experimental.pallas import tpu as pltpu, tpu_sc as plsc


def kernel(x, W):
    B, S, H = x.shape
    info = plsc.get_sparse_core_info()
    NW = info.num_cores * info.num_subcores
    rows_per_w = S // NW
    CH = 32
    n_chunks = rows_per_w // CH
    mesh = plsc.VectorSubcoreMesh(core_axis_name="c", subcore_axis_name="s")

    @functools.partial(
        pl.kernel, mesh=mesh,
        out_type=jax.ShapeDtypeStruct((B, S, H), W.dtype),
        scratch_types=[
            pltpu.VMEM((2, CH, H), W.dtype),
            pltpu.SemaphoreType.DMA,
            pltpu.SemaphoreType.DMA,
        ],
    )
    def body(w_hbm, out_hbm, buf, in_sem, out_sem):
        wid = lax.axis_index("s") * info.num_cores + lax.axis_index("c")
        base = wid * rows_per_w
        first = pltpu.async_copy(w_hbm.at[pl.ds(base, CH)], buf.at[0], in_sem)
        first.wait()
        for c in range(n_chunks):
            slot = c % 2
            off = base + c * CH
            if c + 1 < n_chunks:
                nxt = pltpu.async_copy(
                    w_hbm.at[pl.ds(off + CH, CH)], buf.at[1 - slot], in_sem)
            writes = [
                pltpu.async_copy(buf.at[slot],
                                 out_hbm.at[b, pl.ds(off, CH)], out_sem)
                for b in range(B)
            ]
            for wcp in writes:
                wcp.wait()
            if c + 1 < n_chunks:
                nxt.wait()

    return body(W)
